# baseline (speedup 1.0000x reference)
import functools

import jax
import jax.numpy as jnp
from jax import lax
from jax.experimental import pallas as pl
from jax.experimental.pallas import tpu as pltpu
from jax.experimental.pallas import tpu_sc as plsc

HIDDEN = 768
EPS = 1e-12
L = 16
NVEC = HIDDEN // L
CHUNK = 32
NGROUP = 3
GVEC = NVEC // NGROUP


def _lane_gather(x, idx):
    dnums = lax.GatherDimensionNumbers(
        offset_dims=(), collapsed_slice_dims=(0,), start_index_map=(0,))
    return lax.gather(x, idx[:, None], dnums, slice_sizes=(1,),
                      mode=lax.GatherScatterMode.PROMISE_IN_BOUNDS)


def _allreduce_sum(x):
    idx = lax.iota(jnp.int32, L)
    for sh in (8, 4, 2, 1):
        x = x + _lane_gather(x, jnp.bitwise_xor(idx, sh))
    return x


def _rsqrt_nr(x):
    i = plsc.bitcast(x, jnp.int32)
    i = jnp.int32(0x5F3759DF) - lax.shift_right_logical(i, 1)
    y = plsc.bitcast(i, jnp.float32)
    half = x * 0.5
    for _ in range(2):
        y = y * (1.5 - half * y * y)
    return y


def _make_sc_call(tokens):
    info = plsc.get_sparse_core_info()
    nc, ns = info.num_cores, info.num_subcores
    nw = nc * ns
    tpw = tokens // nw
    nchunks = tpw // CHUNK
    assert nchunks % 2 == 0
    mesh = plsc.VectorSubcoreMesh(core_axis_name="c", subcore_axis_name="s")
    fbuf = jax.ShapeDtypeStruct((CHUNK, HIDDEN), jnp.float32)

    @functools.partial(
        pl.kernel,
        mesh=mesh,
        compiler_params=pltpu.CompilerParams(needs_layout_passes=False),
        out_type=jax.ShapeDtypeStruct((tokens, HIDDEN), jnp.float32),
        scratch_types=[
            pltpu.VMEM((tpw,), jnp.int32),
            pltpu.VMEM((tpw,), jnp.int32),
            pltpu.VMEM(fbuf.shape, jnp.float32),
            pltpu.VMEM(fbuf.shape, jnp.float32),
            pltpu.VMEM(fbuf.shape, jnp.float32),
            pltpu.VMEM(fbuf.shape, jnp.float32),
            pltpu.VMEM((CHUNK, L), jnp.float32),
            pltpu.VMEM((CHUNK, L), jnp.float32),
            pltpu.VMEM((HIDDEN,), jnp.float32),
            pltpu.VMEM((HIDDEN,), jnp.float32),
            pltpu.SemaphoreType.DMA,
            pltpu.SemaphoreType.DMA,
            pltpu.SemaphoreType.DMA,
            pltpu.SemaphoreType.DMA,
        ],
    )
    def sc_kernel(word_hbm, tp_hbm, tok_hbm, tpi_hbm, lnw_hbm, lnb_hbm,
                  out_hbm, tok_v, tpi_v, wb0, tb0, wb1, tb1, mean_b, inv_b,
                  lnw_v, lnb_v, sg0, sg1, so0, so1):
        wid = lax.axis_index("s") * nc + lax.axis_index("c")
        base = wid * tpw
        pltpu.sync_copy(tok_hbm.at[pl.ds(base, tpw)], tok_v)
        pltpu.sync_copy(tpi_hbm.at[pl.ds(base, tpw)], tpi_v)
        pltpu.sync_copy(lnw_hbm, lnw_v)
        pltpu.sync_copy(lnb_hbm, lnb_v)

        def issue_gather(c, wb, tb, sem):
            off = c * CHUNK
            pltpu.async_copy(word_hbm.at[tok_v.at[pl.ds(off, CHUNK)]], wb, sem)
            pltpu.async_copy(tp_hbm.at[tpi_v.at[pl.ds(off, CHUNK)]], tb, sem)

        def drain_gather(wb, tb, sem):
            pltpu.make_async_copy(word_hbm.at[pl.ds(0, CHUNK)], wb, sem).wait()
            pltpu.make_async_copy(word_hbm.at[pl.ds(0, CHUNK)], tb, sem).wait()

        def issue_out(c, wb, sem):
            pltpu.async_copy(wb, out_hbm.at[pl.ds(base + c * CHUNK, CHUNK)], sem)

        def drain_out(wb, sem):
            pltpu.make_async_copy(wb, out_hbm.at[pl.ds(0, CHUNK)], sem).wait()

        def compute(wb, tb):
            def tok_stats(i, carry):
                t0 = 2 * i
                acc = [[jnp.zeros((L,), jnp.float32) for _ in range(2)]
                       for _ in range(2)]
                for j in range(NVEC):
                    sl = pl.ds(j * L, L)
                    for u in range(2):
                        e = wb[t0 + u, sl] + tb[t0 + u, sl]
                        wb[t0 + u, sl] = e
                        acc[u][0] = acc[u][0] + e
                        acc[u][1] = acc[u][1] + e * e
                for u in range(2):
                    mean_v = _allreduce_sum(acc[u][0]) * (1.0 / HIDDEN)
                    var_v = (_allreduce_sum(acc[u][1]) * (1.0 / HIDDEN)
                             - mean_v * mean_v)
                    mean_b[t0 + u, :] = mean_v
                    inv_b[t0 + u, :] = _rsqrt_nr(var_v + EPS)
                return carry

            lax.fori_loop(0, CHUNK // 2, tok_stats, 0)

            for g in range(NGROUP):
                lw = [lnw_v[pl.ds((g * GVEC + k) * L, L)] for k in range(GVEC)]
                lb = [lnb_v[pl.ds((g * GVEC + k) * L, L)] for k in range(GVEC)]

                def tok_norm(i, carry, g=g, lw=lw, lb=lb):
                    t0 = 2 * i
                    mv = [mean_b[t0, :], mean_b[t0 + 1, :]]
                    iv = [inv_b[t0, :], inv_b[t0 + 1, :]]
                    for k in range(GVEC):
                        sl = pl.ds((g * GVEC + k) * L, L)
                        for u in range(2):
                            wb[t0 + u, sl] = ((wb[t0 + u, sl] - mv[u])
                                              * iv[u] * lw[k] + lb[k])
                    return carry

                lax.fori_loop(0, CHUNK // 2, tok_norm, 0)

        issue_gather(0, wb0, tb0, sg0)

        def pair_body(p, carry):
            c0 = 2 * p
            drain_gather(wb0, tb0, sg0)

            @pl.when(p > 0)
            def _():
                drain_out(wb1, so1)

            issue_gather(c0 + 1, wb1, tb1, sg1)
            compute(wb0, tb0)
            issue_out(c0, wb0, so0)
            drain_gather(wb1, tb1, sg1)

            @pl.when(c0 + 2 < nchunks)
            def _():
                drain_out(wb0, so0)
                issue_gather(c0 + 2, wb0, tb0, sg0)
            compute(wb1, tb1)
            issue_out(c0 + 1, wb1, so1)
            return carry

        lax.fori_loop(0, nchunks // 2, pair_body, 0)
        drain_out(wb0, so0)
        drain_out(wb1, so1)

    return sc_kernel


def kernel(token_ids, token_type_ids, position_ids, word_emb, type_emb,
           pos_emb, ln_w, ln_b):
    b, s = token_ids.shape
    tokens = b * s
    max_seq = pos_emb.shape[0]
    tok = token_ids.reshape(-1).astype(jnp.int32)
    tpi = (token_type_ids.astype(jnp.int32) * max_seq
           + position_ids.astype(jnp.int32)).reshape(-1)
    tp_table = (type_emb[:, None, :] + pos_emb[None, :, :]).reshape(-1, HIDDEN)
    out = _make_sc_call(tokens)(word_emb, tp_table, tok, tpi, ln_w, ln_b)
    return out.reshape(b, s, HIDDEN)


# speedup vs baseline: 4.6894x; 4.6894x over previous
import functools

import jax
import jax.numpy as jnp
from jax import lax
from jax.experimental import pallas as pl
from jax.experimental.pallas import tpu as pltpu
from jax.experimental.pallas import tpu_sc as plsc

HIDDEN = 768
EPS = 1e-12
L = 16
NVEC = HIDDEN // L
CHUNK = 32
NGROUP = 3
GVEC = NVEC // NGROUP


def _lane_gather(x, idx):
    dnums = lax.GatherDimensionNumbers(
        offset_dims=(), collapsed_slice_dims=(0,), start_index_map=(0,))
    return lax.gather(x, idx[:, None], dnums, slice_sizes=(1,),
                      mode=lax.GatherScatterMode.PROMISE_IN_BOUNDS)


def _allreduce_sum(x):
    idx = lax.iota(jnp.int32, L)
    for sh in (8, 4, 2, 1):
        x = x + _lane_gather(x, jnp.bitwise_xor(idx, sh))
    return x


def _rsqrt_nr(x):
    i = plsc.bitcast(x, jnp.int32)
    i = jnp.int32(0x5F3759DF) - lax.shift_right_logical(i, 1)
    y = plsc.bitcast(i, jnp.float32)
    half = x * 0.5
    for _ in range(2):
        y = y * (1.5 - half * y * y)
    return y


def _make_sc_call(tokens):
    info = plsc.get_sparse_core_info()
    nc, ns = info.num_cores, info.num_subcores
    nw = nc * ns
    tpw = tokens // nw
    nchunks = tpw // CHUNK
    assert nchunks % 2 == 0
    mesh = plsc.VectorSubcoreMesh(core_axis_name="c", subcore_axis_name="s")
    fbuf = jax.ShapeDtypeStruct((CHUNK, HIDDEN), jnp.float32)

    @functools.partial(
        pl.kernel,
        mesh=mesh,
        compiler_params=pltpu.CompilerParams(needs_layout_passes=False),
        out_type=jax.ShapeDtypeStruct((tokens, HIDDEN), jnp.float32),
        scratch_types=[
            pltpu.VMEM((tpw,), jnp.int32),
            pltpu.VMEM((tpw,), jnp.int32),
            pltpu.VMEM(fbuf.shape, jnp.float32),
            pltpu.VMEM(fbuf.shape, jnp.float32),
            pltpu.VMEM(fbuf.shape, jnp.float32),
            pltpu.VMEM(fbuf.shape, jnp.float32),
            pltpu.VMEM((CHUNK, L), jnp.float32),
            pltpu.VMEM((CHUNK, L), jnp.float32),
            pltpu.VMEM((HIDDEN,), jnp.float32),
            pltpu.VMEM((HIDDEN,), jnp.float32),
            pltpu.SemaphoreType.DMA,
            pltpu.SemaphoreType.DMA,
            pltpu.SemaphoreType.DMA,
            pltpu.SemaphoreType.DMA,
        ],
    )
    def sc_kernel(word_hbm, tp_hbm, tok_hbm, tpi_hbm, lnw_hbm, lnb_hbm,
                  out_hbm, tok_v, tpi_v, wb0, tb0, wb1, tb1, mean_b, inv_b,
                  lnw_v, lnb_v, sg0, sg1, so0, so1):
        wid = lax.axis_index("s") * nc + lax.axis_index("c")
        base = wid * tpw
        pltpu.sync_copy(tok_hbm.at[pl.ds(base, tpw)], tok_v)
        pltpu.sync_copy(tpi_hbm.at[pl.ds(base, tpw)], tpi_v)
        pltpu.sync_copy(lnw_hbm, lnw_v)
        pltpu.sync_copy(lnb_hbm, lnb_v)

        def issue_gather(c, wb, tb, sem):
            off = c * CHUNK
            pltpu.async_copy(word_hbm.at[tok_v.at[pl.ds(off, CHUNK)]], wb, sem)
            pltpu.async_copy(tp_hbm.at[tpi_v.at[pl.ds(off, CHUNK)]], tb, sem)

        def drain_gather(wb, tb, sem):
            pltpu.make_async_copy(word_hbm.at[pl.ds(0, CHUNK)], wb, sem).wait()
            pltpu.make_async_copy(word_hbm.at[pl.ds(0, CHUNK)], tb, sem).wait()

        def issue_out(c, wb, sem):
            pltpu.async_copy(wb, out_hbm.at[pl.ds(base + c * CHUNK, CHUNK)], sem)

        def drain_out(wb, sem):
            pltpu.make_async_copy(wb, out_hbm.at[pl.ds(0, CHUNK)], sem).wait()

        def compute(wb, tb):
            @plsc.parallel_loop(0, CHUNK, step=1, unroll=2)
            def tok_stats(t):
                acc_s = jnp.zeros((L,), jnp.float32)
                acc_q = jnp.zeros((L,), jnp.float32)
                for j in range(NVEC):
                    sl = pl.ds(j * L, L)
                    e = wb[t, sl] + tb[t, sl]
                    wb[t, sl] = e
                    acc_s = acc_s + e
                    acc_q = acc_q + e * e
                mean_v = _allreduce_sum(acc_s) * (1.0 / HIDDEN)
                var_v = _allreduce_sum(acc_q) * (1.0 / HIDDEN) - mean_v * mean_v
                mean_b[t, :] = mean_v
                inv_b[t, :] = _rsqrt_nr(var_v + EPS)

            for g in range(NGROUP):
                lw = [lnw_v[pl.ds((g * GVEC + k) * L, L)] for k in range(GVEC)]
                lb = [lnb_v[pl.ds((g * GVEC + k) * L, L)] for k in range(GVEC)]

                @plsc.parallel_loop(0, CHUNK, step=1, unroll=2)
                def tok_norm(t, g=g, lw=lw, lb=lb):
                    mean_v = mean_b[t, :]
                    inv_v = inv_b[t, :]
                    for k in range(GVEC):
                        sl = pl.ds((g * GVEC + k) * L, L)
                        wb[t, sl] = (wb[t, sl] - mean_v) * inv_v * lw[k] + lb[k]

        issue_gather(0, wb0, tb0, sg0)

        def pair_body(p, carry):
            c0 = 2 * p
            drain_gather(wb0, tb0, sg0)

            @pl.when(p > 0)
            def _():
                drain_out(wb1, so1)

            issue_gather(c0 + 1, wb1, tb1, sg1)
            compute(wb0, tb0)
            issue_out(c0, wb0, so0)
            drain_gather(wb1, tb1, sg1)

            @pl.when(c0 + 2 < nchunks)
            def _():
                drain_out(wb0, so0)
                issue_gather(c0 + 2, wb0, tb0, sg0)
            compute(wb1, tb1)
            issue_out(c0 + 1, wb1, so1)
            return carry

        lax.fori_loop(0, nchunks // 2, pair_body, 0)
        drain_out(wb0, so0)
        drain_out(wb1, so1)

    return sc_kernel


def kernel(token_ids, token_type_ids, position_ids, word_emb, type_emb,
           pos_emb, ln_w, ln_b):
    b, s = token_ids.shape
    tokens = b * s
    max_seq = pos_emb.shape[0]
    tok = token_ids.reshape(-1).astype(jnp.int32)
    tpi = (token_type_ids.astype(jnp.int32) * max_seq
           + position_ids.astype(jnp.int32)).reshape(-1)
    tp_table = (type_emb[:, None, :] + pos_emb[None, :, :]).reshape(-1, HIDDEN)
    out = _make_sc_call(tokens)(word_emb, tp_table, tok, tpi, ln_w, ln_b)
    return out.reshape(b, s, HIDDEN)
